# baseline (speedup 1.0000x reference)
import functools

import jax
import jax.numpy as jnp
from jax import lax
from jax.experimental import pallas as pl
from jax.experimental.pallas import tpu as pltpu
from jax.experimental.pallas import tpu_sc as plsc

VOCAB_N = 100000
EMB_D = 64
BATCH_N = 4096
CTX_L = 20

NUM_CORES = 2
NUM_SUBCORES = 16
NUM_WORKERS = NUM_CORES * NUM_SUBCORES
ROWS_PER_W = BATCH_N // NUM_WORKERS
LANES = 16
COL_CHUNKS = EMB_D // LANES


def _sc_gather_mean(idx_t, emb_table):
    mesh = plsc.VectorSubcoreMesh(core_axis_name="c", subcore_axis_name="s")

    @functools.partial(
        pl.kernel,
        mesh=mesh,
        compiler_params=pltpu.CompilerParams(use_tc_tiling_on_sc=False),
        out_type=jax.ShapeDtypeStruct((BATCH_N, EMB_D), jnp.float32),
        scratch_types=[
            pltpu.VMEM((CTX_L, ROWS_PER_W), jnp.int32),
            pltpu.VMEM((ROWS_PER_W, EMB_D), jnp.float32),
            pltpu.VMEM((ROWS_PER_W, EMB_D), jnp.float32),
            pltpu.SemaphoreType.DMA,
        ],
    )
    def run(idx_hbm, emb_hbm, out_hbm, idx_v, buf_v, acc_v, sem):
        wid = lax.axis_index("s") * NUM_CORES + lax.axis_index("c")
        base = wid * ROWS_PER_W
        pltpu.sync_copy(idx_hbm.at[:, pl.ds(base, ROWS_PER_W)], idx_v)

        def zero_row(i, carry):
            for c in range(COL_CHUNKS):
                acc_v[i, pl.ds(c * LANES, LANES)] = jnp.zeros((LANES,), jnp.float32)
            return carry

        lax.fori_loop(0, ROWS_PER_W, zero_row, 0)

        def ctx_step(j, carry):
            pltpu.async_copy(emb_hbm.at[idx_v.at[j]], buf_v, sem).wait()

            def row(i, inner):
                for c in range(COL_CHUNKS):
                    sl = pl.ds(c * LANES, LANES)
                    acc_v[i, sl] += buf_v[i, sl]
                return inner

            lax.fori_loop(0, ROWS_PER_W, row, 0)
            return carry

        lax.fori_loop(0, CTX_L, ctx_step, 0)

        def scale_row(i, carry):
            for c in range(COL_CHUNKS):
                sl = pl.ds(c * LANES, LANES)
                acc_v[i, sl] = acc_v[i, sl] * (1.0 / CTX_L)
            return carry

        lax.fori_loop(0, ROWS_PER_W, scale_row, 0)
        pltpu.sync_copy(acc_v, out_hbm.at[pl.ds(base, ROWS_PER_W)])

    return run(idx_t, emb_table)


def _tc_matmul_bias(avg, w, b2):
    tile_b, tile_v = 2048, 512
    grid = (BATCH_N // tile_b, pl.cdiv(VOCAB_N, tile_v))

    def body(avg_ref, w_ref, b_ref, out_ref):
        out_ref[...] = lax.dot_general(
            avg_ref[...], w_ref[...],
            (((1,), (1,)), ((), ())),
            preferred_element_type=jnp.float32,
        ) + b_ref[...]

    return pl.pallas_call(
        body,
        grid=grid,
        in_specs=[
            pl.BlockSpec((tile_b, EMB_D), lambda i, j: (i, 0)),
            pl.BlockSpec((tile_v, EMB_D), lambda i, j: (j, 0)),
            pl.BlockSpec((1, tile_v), lambda i, j: (0, j)),
        ],
        out_specs=pl.BlockSpec((tile_b, tile_v), lambda i, j: (i, j)),
        out_shape=jax.ShapeDtypeStruct((BATCH_N, VOCAB_N), jnp.float32),
    )(avg, w, b2)


def kernel(context_indices, emb_table, W, b):
    idx_t = context_indices.astype(jnp.int32).T
    avg = _sc_gather_mean(idx_t, emb_table)
    return _tc_matmul_bias(avg, W, b.reshape(1, VOCAB_N))


# speedup vs baseline: 1.0678x; 1.0678x over previous
import functools

import jax
import jax.numpy as jnp
from jax import lax
from jax.experimental import pallas as pl
from jax.experimental.pallas import tpu as pltpu
from jax.experimental.pallas import tpu_sc as plsc

VOCAB_N = 100000
EMB_D = 64
BATCH_N = 4096
CTX_L = 20

NUM_CORES = 2
NUM_SUBCORES = 16
NUM_WORKERS = NUM_CORES * NUM_SUBCORES
ROWS_PER_W = BATCH_N // NUM_WORKERS
LANES = 16
COL_CHUNKS = EMB_D // LANES


def _sc_gather_mean(idx_t, emb_table):
    mesh = plsc.VectorSubcoreMesh(core_axis_name="c", subcore_axis_name="s")

    n_groups = 4
    gsz = CTX_L // n_groups

    @functools.partial(
        pl.kernel,
        mesh=mesh,
        compiler_params=pltpu.CompilerParams(use_tc_tiling_on_sc=False),
        out_type=jax.ShapeDtypeStruct((BATCH_N, EMB_D), jnp.float32),
        scratch_types=[
            pltpu.VMEM((CTX_L, ROWS_PER_W), jnp.int32),
            pltpu.VMEM((2, gsz, ROWS_PER_W, EMB_D), jnp.float32),
            pltpu.VMEM((ROWS_PER_W, EMB_D), jnp.float32),
            pltpu.SemaphoreType.DMA,
            pltpu.SemaphoreType.DMA,
        ],
    )
    def run(idx_hbm, emb_hbm, out_hbm, idx_v, buf_v, acc_v, sem0, sem1):
        wid = lax.axis_index("s") * NUM_CORES + lax.axis_index("c")
        base = wid * ROWS_PER_W
        pltpu.sync_copy(idx_hbm.at[:, pl.ds(base, ROWS_PER_W)], idx_v)
        sems = (sem0, sem1)

        def fire(g):
            copies = []
            for j in range(gsz):
                copies.append(pltpu.async_copy(
                    emb_hbm.at[idx_v.at[g * gsz + j]],
                    buf_v.at[g % 2, j], sems[g % 2]))
            return copies

        pending = fire(0)
        for g in range(n_groups):
            for cp in pending:
                cp.wait()
            if g + 1 < n_groups:
                pending = fire(g + 1)

            def row(i, carry, g=g):
                for c in range(COL_CHUNKS):
                    sl = pl.ds(c * LANES, LANES)
                    s = buf_v[g % 2, 0, i, sl]
                    for j in range(1, gsz):
                        s = s + buf_v[g % 2, j, i, sl]
                    if g == 0:
                        acc_v[i, sl] = s
                    elif g == n_groups - 1:
                        acc_v[i, sl] = (acc_v[i, sl] + s) * (1.0 / CTX_L)
                    else:
                        acc_v[i, sl] = acc_v[i, sl] + s
                return carry

            lax.fori_loop(0, ROWS_PER_W, row, 0)

        pltpu.sync_copy(acc_v, out_hbm.at[pl.ds(base, ROWS_PER_W)])

    return run(idx_t, emb_table)


def _tc_matmul_bias(avg, w, b2):
    tile_b, tile_v = 4096, 1024
    grid = (BATCH_N // tile_b, pl.cdiv(VOCAB_N, tile_v))

    def body(avg_ref, w_ref, b_ref, out_ref):
        out_ref[...] = lax.dot_general(
            avg_ref[...], w_ref[...],
            (((1,), (1,)), ((), ())),
            preferred_element_type=jnp.float32,
        ) + b_ref[...]

    return pl.pallas_call(
        body,
        grid=grid,
        in_specs=[
            pl.BlockSpec((tile_b, EMB_D), lambda i, j: (i, 0)),
            pl.BlockSpec((tile_v, EMB_D), lambda i, j: (j, 0)),
            pl.BlockSpec((1, tile_v), lambda i, j: (0, j)),
        ],
        out_specs=pl.BlockSpec((tile_b, tile_v), lambda i, j: (i, j)),
        out_shape=jax.ShapeDtypeStruct((BATCH_N, VOCAB_N), jnp.float32),
    )(avg, w, b2)


def kernel(context_indices, emb_table, W, b):
    idx_t = context_indices.astype(jnp.int32).T
    avg = _sc_gather_mean(idx_t, emb_table)
    return _tc_matmul_bias(avg, W, b.reshape(1, VOCAB_N))


# speedup vs baseline: 3.4041x; 3.1880x over previous
import functools

import jax
import jax.numpy as jnp
from jax import lax
from jax.experimental import pallas as pl
from jax.experimental.pallas import tpu as pltpu
from jax.experimental.pallas import tpu_sc as plsc

VOCAB_N = 100000
EMB_D = 64
BATCH_N = 4096
CTX_L = 20

NUM_CORES = 2
NUM_SUBCORES = 16
NUM_WORKERS = NUM_CORES * NUM_SUBCORES
ROWS_PER_W = BATCH_N // NUM_WORKERS
LANES = 16
COL_CHUNKS = EMB_D // LANES


def _sc_gather_mean(idx_t, emb_table):
    mesh = plsc.VectorSubcoreMesh(core_axis_name="c", subcore_axis_name="s")

    n_groups = 4
    gsz = CTX_L // n_groups

    @functools.partial(
        pl.kernel,
        mesh=mesh,
        compiler_params=pltpu.CompilerParams(use_tc_tiling_on_sc=False),
        out_type=jax.ShapeDtypeStruct((BATCH_N, EMB_D), jnp.float32),
        scratch_types=[
            pltpu.VMEM((CTX_L, ROWS_PER_W), jnp.int32),
            pltpu.VMEM((2, gsz, ROWS_PER_W, EMB_D), jnp.float32),
            pltpu.VMEM((ROWS_PER_W, EMB_D), jnp.float32),
            pltpu.SemaphoreType.DMA,
            pltpu.SemaphoreType.DMA,
        ],
    )
    def run(idx_hbm, emb_hbm, out_hbm, idx_v, buf_v, acc_v, sem0, sem1):
        wid = lax.axis_index("s") * NUM_CORES + lax.axis_index("c")
        base = wid * ROWS_PER_W
        pltpu.sync_copy(idx_hbm.at[:, pl.ds(base, ROWS_PER_W)], idx_v)
        sems = (sem0, sem1)

        def fire(g):
            copies = []
            for j in range(gsz):
                copies.append(pltpu.async_copy(
                    emb_hbm.at[idx_v.at[g * gsz + j]],
                    buf_v.at[g % 2, j], sems[g % 2]))
            return copies

        pending = fire(0)
        for g in range(n_groups):
            for cp in pending:
                cp.wait()
            if g + 1 < n_groups:
                pending = fire(g + 1)

            def row(i, carry, g=g):
                for c in range(COL_CHUNKS):
                    sl = pl.ds(c * LANES, LANES)
                    s = buf_v[g % 2, 0, i, sl]
                    for j in range(1, gsz):
                        s = s + buf_v[g % 2, j, i, sl]
                    if g == 0:
                        acc_v[i, sl] = s
                    elif g == n_groups - 1:
                        acc_v[i, sl] = (acc_v[i, sl] + s) * (1.0 / CTX_L)
                    else:
                        acc_v[i, sl] = acc_v[i, sl] + s
                return carry

            lax.fori_loop(0, ROWS_PER_W, row, 0)

        pltpu.sync_copy(acc_v, out_hbm.at[pl.ds(base, ROWS_PER_W)])

    return run(idx_t, emb_table)


def _tc_matmul_bias_t(avg, w_t, b2):
    tile_v = 1024
    grid = (pl.cdiv(VOCAB_N, tile_v),)

    def body(wt_ref, avg_ref, b_ref, out_ref):
        out_ref[...] = lax.dot_general(
            wt_ref[...], avg_ref[...],
            (((0,), (1,)), ((), ())),
            preferred_element_type=jnp.float32,
        ) + b_ref[...]

    return pl.pallas_call(
        body,
        grid=grid,
        in_specs=[
            pl.BlockSpec((EMB_D, tile_v), lambda j: (0, j)),
            pl.BlockSpec((BATCH_N, EMB_D), lambda j: (0, 0)),
            pl.BlockSpec((tile_v, 1), lambda j: (j, 0)),
        ],
        out_specs=pl.BlockSpec((tile_v, BATCH_N), lambda j: (j, 0)),
        out_shape=jax.ShapeDtypeStruct((VOCAB_N, BATCH_N), jnp.float32),
    )(w_t, avg, b2)


def kernel(context_indices, emb_table, W, b):
    idx_t = context_indices.astype(jnp.int32).T
    avg = _sc_gather_mean(idx_t, emb_table)
    out_t = _tc_matmul_bias_t(avg, W.T, b.reshape(VOCAB_N, 1))
    return out_t.T


# speedup vs baseline: 3.4849x; 1.0238x over previous
import functools

import jax
import jax.numpy as jnp
from jax import lax
from jax.experimental import pallas as pl
from jax.experimental.pallas import tpu as pltpu
from jax.experimental.pallas import tpu_sc as plsc

VOCAB_N = 100000
EMB_D = 64
BATCH_N = 4096
CTX_L = 20

VROWS = 782
VPAD = VROWS * 128

NUM_CORES = 2
NUM_SUBCORES = 16
NUM_WORKERS = NUM_CORES * NUM_SUBCORES
DIMS_PER_W = EMB_D // NUM_WORKERS
LANES = 16
BCHUNK = 512
N_BCHUNKS = BATCH_N // BCHUNK


def _sc_gather_mean_t(idx_t, emb_3d):
    mesh = plsc.VectorSubcoreMesh(core_axis_name="c", subcore_axis_name="s")

    @functools.partial(
        pl.kernel,
        mesh=mesh,
        compiler_params=pltpu.CompilerParams(
            use_tc_tiling_on_sc=False, needs_layout_passes=False),
        out_type=jax.ShapeDtypeStruct((EMB_D, BATCH_N), jnp.float32),
        scratch_types=[
            pltpu.VMEM((VROWS, 128), jnp.float32),
            pltpu.VMEM((2 * CTX_L, BCHUNK), jnp.int32),
            pltpu.VMEM((BCHUNK,), jnp.float32),
            pltpu.SemaphoreType.DMA,
            pltpu.SemaphoreType.DMA,
            pltpu.SemaphoreType.DMA,
        ],
    )
    def run(idx_hbm, emb_hbm, out_hbm, row_v, idx_v, avg_v, sem_r, sem_i, sem_o):
        wid = lax.axis_index("s") * NUM_CORES + lax.axis_index("c")

        for d in range(DIMS_PER_W):
            e = wid * DIMS_PER_W + d
            row_cp = pltpu.async_copy(emb_hbm.at[e], row_v, sem_r)
            idx_cp = pltpu.async_copy(
                idx_hbm.at[:, pl.ds(0, BCHUNK)],
                idx_v.at[pl.ds(0, CTX_L)], sem_i)
            row_cp.wait()

            for q in range(N_BCHUNKS):
                idx_cp.wait()
                if q + 1 < N_BCHUNKS:
                    idx_cp = pltpu.async_copy(
                        idx_hbm.at[:, pl.ds((q + 1) * BCHUNK, BCHUNK)],
                        idx_v.at[pl.ds(((q + 1) % 2) * CTX_L, CTX_L)], sem_i)

                def chunk_body(i, carry, q=q):
                    b0 = i * LANES
                    jbase = (q % 2) * CTX_L

                    def ctx_step(j, acc):
                        iv = idx_v[jbase + j, pl.ds(b0, LANES)]
                        return acc + plsc.load_gather(
                            row_v, [iv >> 7, iv & 127])

                    acc = lax.fori_loop(
                        0, CTX_L, ctx_step, jnp.zeros((LANES,), jnp.float32))
                    avg_v[pl.ds(b0, LANES)] = acc * (1.0 / CTX_L)
                    return carry

                lax.fori_loop(0, BCHUNK // LANES, chunk_body, 0)
                pltpu.async_copy(
                    avg_v, out_hbm.at[e, pl.ds(q * BCHUNK, BCHUNK)], sem_o
                ).wait()

    return run(idx_t, emb_3d)


def _tc_matmul_bias_t(avg_t, w_t, b2):
    tile_v = 1024
    grid = (pl.cdiv(VOCAB_N, tile_v),)

    def body(wt_ref, avg_ref, b_ref, out_ref):
        out_ref[...] = lax.dot_general(
            wt_ref[...], avg_ref[...],
            (((0,), (0,)), ((), ())),
            preferred_element_type=jnp.float32,
        ) + b_ref[...]

    return pl.pallas_call(
        body,
        grid=grid,
        in_specs=[
            pl.BlockSpec((EMB_D, tile_v), lambda j: (0, j)),
            pl.BlockSpec((EMB_D, BATCH_N), lambda j: (0, 0)),
            pl.BlockSpec((tile_v, 1), lambda j: (j, 0)),
        ],
        out_specs=pl.BlockSpec((tile_v, BATCH_N), lambda j: (j, 0)),
        out_shape=jax.ShapeDtypeStruct((VOCAB_N, BATCH_N), jnp.float32),
    )(w_t, avg_t, b2)


def kernel(context_indices, emb_table, W, b):
    idx_t = context_indices.astype(jnp.int32).T
    emb_3d = jnp.pad(
        emb_table.T, ((0, 0), (0, VPAD - VOCAB_N))).reshape(EMB_D, VROWS, 128)
    avg_t = _sc_gather_mean_t(idx_t, emb_3d)
    out_t = _tc_matmul_bias_t(avg_t, W.T, b.reshape(VOCAB_N, 1))
    return out_t.T


# speedup vs baseline: 3.4886x; 1.0010x over previous
import functools

import jax
import jax.numpy as jnp
from jax import lax
from jax.experimental import pallas as pl
from jax.experimental.pallas import tpu as pltpu
from jax.experimental.pallas import tpu_sc as plsc

VOCAB_N = 100000
EMB_D = 64
BATCH_N = 4096
CTX_L = 20

VROWS = 782
VPAD = VROWS * 128

NUM_CORES = 2
NUM_SUBCORES = 16
NUM_WORKERS = NUM_CORES * NUM_SUBCORES
DIMS_PER_W = EMB_D // NUM_WORKERS
LANES = 16
BCHUNK = 512
N_BCHUNKS = BATCH_N // BCHUNK


def _sc_gather_mean_t(idx_t, emb_t):
    mesh = plsc.VectorSubcoreMesh(core_axis_name="c", subcore_axis_name="s")

    @functools.partial(
        pl.kernel,
        mesh=mesh,
        compiler_params=pltpu.CompilerParams(
            use_tc_tiling_on_sc=False, needs_layout_passes=False),
        out_type=jax.ShapeDtypeStruct((EMB_D, BATCH_N), jnp.float32),
        scratch_types=[
            pltpu.VMEM((VOCAB_N,), jnp.float32),
            pltpu.VMEM((2 * CTX_L, BCHUNK), jnp.int32),
            pltpu.VMEM((BCHUNK,), jnp.float32),
            pltpu.SemaphoreType.DMA,
            pltpu.SemaphoreType.DMA,
            pltpu.SemaphoreType.DMA,
        ],
    )
    def run(idx_hbm, emb_hbm, out_hbm, row_v, idx_v, avg_v, sem_r, sem_i, sem_o):
        wid = lax.axis_index("s") * NUM_CORES + lax.axis_index("c")

        for d in range(DIMS_PER_W):
            e = wid * DIMS_PER_W + d
            row_cp = pltpu.async_copy(emb_hbm.at[e], row_v, sem_r)
            idx_cp = pltpu.async_copy(
                idx_hbm.at[:, pl.ds(0, BCHUNK)],
                idx_v.at[pl.ds(0, CTX_L)], sem_i)
            row_cp.wait()

            for q in range(N_BCHUNKS):
                idx_cp.wait()
                if q + 1 < N_BCHUNKS:
                    idx_cp = pltpu.async_copy(
                        idx_hbm.at[:, pl.ds((q + 1) * BCHUNK, BCHUNK)],
                        idx_v.at[pl.ds(((q + 1) % 2) * CTX_L, CTX_L)], sem_i)

                def chunk_body(i, carry, q=q):
                    b0 = i * LANES
                    jbase = (q % 2) * CTX_L

                    def ctx_step(j, acc):
                        iv = idx_v[jbase + j, pl.ds(b0, LANES)]
                        return acc + plsc.load_gather(row_v, [iv])

                    acc = lax.fori_loop(
                        0, CTX_L, ctx_step, jnp.zeros((LANES,), jnp.float32))
                    avg_v[pl.ds(b0, LANES)] = acc * (1.0 / CTX_L)
                    return carry

                lax.fori_loop(0, BCHUNK // LANES, chunk_body, 0)
                pltpu.async_copy(
                    avg_v, out_hbm.at[e, pl.ds(q * BCHUNK, BCHUNK)], sem_o
                ).wait()

    return run(idx_t, emb_t)


def _tc_matmul_bias_t(avg_t, w_t, b2):
    tile_v = 1024
    grid = (pl.cdiv(VOCAB_N, tile_v),)

    def body(wt_ref, avg_ref, b_ref, out_ref):
        out_ref[...] = lax.dot_general(
            wt_ref[...], avg_ref[...],
            (((0,), (0,)), ((), ())),
            preferred_element_type=jnp.float32,
        ) + b_ref[...]

    return pl.pallas_call(
        body,
        grid=grid,
        in_specs=[
            pl.BlockSpec((EMB_D, tile_v), lambda j: (0, j)),
            pl.BlockSpec((EMB_D, BATCH_N), lambda j: (0, 0)),
            pl.BlockSpec((tile_v, 1), lambda j: (j, 0)),
        ],
        out_specs=pl.BlockSpec((tile_v, BATCH_N), lambda j: (j, 0)),
        out_shape=jax.ShapeDtypeStruct((VOCAB_N, BATCH_N), jnp.float32),
    )(w_t, avg_t, b2)


def kernel(context_indices, emb_table, W, b):
    idx_t = context_indices.astype(jnp.int32).T
    avg_t = _sc_gather_mean_t(idx_t, emb_table.T)
    out_t = _tc_matmul_bias_t(avg_t, W.T, b.reshape(VOCAB_N, 1))
    return out_t.T
